# Initial kernel scaffold; baseline (speedup 1.0000x reference)
import jax
import jax.numpy as jnp
from jax import lax
from jax.experimental import pallas as pl
from jax.experimental.pallas import tpu as pltpu
from jax.experimental.pallas import tpu_sc as plsc

H = 128
G = 64
TW = 144
CHUNK = 128
NC, NS = 2, 16
NW = NC * NS
BN = 512
BE = 1024
NEG = -3.4e38


def _rup(a, b):
    return (a + b - 1) // b * b


def _silu(v):
    return v * jax.nn.sigmoid(v)


def _dot(a, b):
    return jnp.dot(a, b, preferred_element_type=jnp.float32)


def _full(shape):
    return pl.BlockSpec(shape, lambda i: (0,) * len(shape))


def _gather(td, ts, dig, sig, e_pad):
    epw = e_pad // NW
    nch = epw // CHUNK
    mesh = plsc.VectorSubcoreMesh(core_axis_name="c", subcore_axis_name="s",
                                  num_cores=NC, num_subcores=NS)

    def body(td_h, ts_h, di_h, si_h, gd_h, gs_h, di_v, si_v, rd_v, rs_v, s1, s2):
        wid = lax.axis_index("s") * NC + lax.axis_index("c")
        base = wid * epw

        def chunk(i, carry):
            off = base + i * CHUNK
            pltpu.sync_copy(di_h.at[pl.ds(off, CHUNK)], di_v)
            pltpu.sync_copy(si_h.at[pl.ds(off, CHUNK)], si_v)
            c1 = pltpu.async_copy(td_h.at[di_v], rd_v, s1)
            c2 = pltpu.async_copy(ts_h.at[si_v], rs_v, s2)
            c1.wait()
            c2.wait()
            pltpu.sync_copy(rd_v, gd_h.at[pl.ds(off, CHUNK)])
            pltpu.sync_copy(rs_v, gs_h.at[pl.ds(off, CHUNK)])
            return carry

        lax.fori_loop(0, nch, chunk, 0)

    call = pl.kernel(
        body,
        out_type=(jax.ShapeDtypeStruct((e_pad, TW), jnp.float32),
                  jax.ShapeDtypeStruct((e_pad, TW), jnp.float32)),
        mesh=mesh,
        scratch_types=[
            pltpu.VMEM((CHUNK,), jnp.int32),
            pltpu.VMEM((CHUNK,), jnp.int32),
            pltpu.VMEM((CHUNK, TW), jnp.float32),
            pltpu.VMEM((CHUNK, TW), jnp.float32),
            pltpu.SemaphoreType.DMA,
            pltpu.SemaphoreType.DMA,
        ],
    )
    return call(td, ts, dig, sig)


def _scatter(s_arr, dsc, ztile, e_pad, n_pad):
    epw = e_pad // NW
    nch = epw // CHUNK
    per_tile = n_pad // NS
    nzc = per_tile // CHUNK
    mesh = plsc.VectorSubcoreMesh(core_axis_name="c", subcore_axis_name="s",
                                  num_cores=NC, num_subcores=NS)

    def body(s_h, idx_h, z_h, out_h, idx_v, rows_v, acc_sh):
        c = lax.axis_index("c")
        s = lax.axis_index("s")
        wid = s * NC + c
        pltpu.sync_copy(z_h, rows_v)
        for k in range(nzc):
            pltpu.sync_copy(rows_v,
                            acc_sh.at[pl.ds(s * per_tile + k * CHUNK, CHUNK)])
        plsc.subcore_barrier()

        def chunk(i, carry):
            off = wid * epw + i * CHUNK
            pltpu.sync_copy(idx_h.at[pl.ds(off, CHUNK)], idx_v)
            pltpu.sync_copy(s_h.at[pl.ds(off, CHUNK)], rows_v)
            pltpu.sync_copy(rows_v, acc_sh.at[idx_v], add=True)
            return carry

        lax.fori_loop(0, nch, chunk, 0)
        plsc.subcore_barrier()
        for k in range(nzc):
            rof = s * per_tile + k * CHUNK
            pltpu.sync_copy(acc_sh.at[pl.ds(rof, CHUNK)], rows_v)
            pltpu.sync_copy(rows_v, out_h.at[c, pl.ds(rof, CHUNK)])

    call = pl.kernel(
        body,
        out_type=jax.ShapeDtypeStruct((NC, n_pad, TW), jnp.float32),
        mesh=mesh,
        scratch_types=[
            pltpu.VMEM((CHUNK,), jnp.int32),
            pltpu.VMEM((CHUNK, TW), jnp.float32),
            pltpu.VMEM_SHARED((n_pad, TW), jnp.float32),
        ],
    )
    return call(s_arr, dsc, ztile)


def _entry(xp, posp, curvp, w_in, b_in, wa, wb, wcd, wcs, n_pad):
    nin = xp.shape[1]
    cd = curvp.shape[1]

    def body(x_r, pos_r, curv_r, win_r, bin_r, wa_r, wb_r, wcd_r, wcs_r,
             h_r, td_r, ts_r):
        h = _silu(_dot(x_r[...], win_r[...]) + bin_r[...])
        a = _dot(h, wa_r[...])
        b = _dot(h, wb_r[...])
        ccd = _dot(curv_r[...], wcd_r[...])
        ccs = _dot(curv_r[...], wcs_r[...])
        z = jnp.zeros((BN, TW - H - 4), jnp.float32)
        pos = pos_r[...]
        h_r[...] = h
        td_r[...] = jnp.concatenate([a, pos, ccd, z], axis=1)
        ts_r[...] = jnp.concatenate([b, -pos, ccs, z], axis=1)

    return pl.pallas_call(
        body,
        grid=(n_pad // BN,),
        in_specs=[
            pl.BlockSpec((BN, nin), lambda i: (i, 0)),
            pl.BlockSpec((BN, 3), lambda i: (i, 0)),
            pl.BlockSpec((BN, cd), lambda i: (i, 0)),
            _full((nin, H)), _full((1, H)), _full((H, H)), _full((H, H)),
            _full((cd, 1)), _full((cd, 1)),
        ],
        out_specs=[
            pl.BlockSpec((BN, H), lambda i: (i, 0)),
            pl.BlockSpec((BN, TW), lambda i: (i, 0)),
            pl.BlockSpec((BN, TW), lambda i: (i, 0)),
        ],
        out_shape=[
            jax.ShapeDtypeStruct((n_pad, H), jnp.float32),
            jax.ShapeDtypeStruct((n_pad, TW), jnp.float32),
            jax.ShapeDtypeStruct((n_pad, TW), jnp.float32),
        ],
    )(xp, posp, curvp, w_in, b_in, wa, wb, wcd, wcs)


def _edge(gd, gs, we2, wx, wd2, wcw, be1, be2, bc, bx, e_pad):
    def body(gd_r, gs_r, we2_r, wx_r, wd2_r, wcw_r, be1_r, be2_r, bc_r, bx_r,
             out_r):
        gdv = gd_r[...]
        gsv = gs_r[...]
        pre = gdv[:, :H] + gsv[:, :H]
        diff = gdv[:, H:H + 3] + gsv[:, H:H + 3]
        scw = gdv[:, H + 3:H + 4] + gsv[:, H + 3:H + 4]
        d2 = jnp.sum(diff * diff, axis=1, keepdims=True)
        cw = jax.nn.sigmoid(scw + bc_r[0, 0])
        m1 = _silu(pre + d2 * wd2_r[...] + cw * wcw_r[...] + be1_r[...])
        m2 = _silu(_dot(m1, we2_r[...]) + be2_r[...]) * cw
        coef = jnp.tanh(_dot(m2, wx_r[...]) + bx_r[0, 0])
        dx = diff * coef
        ones = jnp.ones((BE, 1), jnp.float32)
        z = jnp.zeros((BE, TW - H - 4), jnp.float32)
        out_r[...] = jnp.concatenate([m2, dx, ones, z], axis=1)

    return pl.pallas_call(
        body,
        grid=(e_pad // BE,),
        in_specs=[
            pl.BlockSpec((BE, TW), lambda i: (i, 0)),
            pl.BlockSpec((BE, TW), lambda i: (i, 0)),
            _full((H, H)), _full((H, 1)), _full((1, H)), _full((1, H)),
            _full((1, H)), _full((1, H)), _full((1, 1)), _full((1, 1)),
        ],
        out_specs=pl.BlockSpec((BE, TW), lambda i: (i, 0)),
        out_shape=jax.ShapeDtypeStruct((e_pad, TW), jnp.float32),
    )(gd, gs, we2, wx, wd2, wcw, be1, be2, bc, bx)


def _node(l, last, h, acc, xo, deg, curvp, wh1a, wh1b, bh1, wh2, bh2,
          wa, wb, wcd, wcs, n_pad):
    cd = curvp.shape[1]

    def body(*refs):
        it = iter(refs)
        h_r = next(it)
        acc_r = next(it)
        xo_r = next(it)
        deg_r = None if l == 0 else next(it)
        if not last:
            curv_r = next(it)
            wa_r = next(it)
            wb_r = next(it)
            wcd_r = next(it)
            wcs_r = next(it)
        wh1a_r = next(it)
        wh1b_r = next(it)
        bh1_r = next(it)
        wh2_r = next(it)
        bh2_r = next(it)
        ho_r = next(it)
        xoo_r = next(it)
        dego_r = next(it) if l == 0 else None
        if not last:
            td_r = next(it)
            ts_r = next(it)

        a0 = acc_r[0]
        a1 = acc_r[1]
        agg = a0[:, :H] + a1[:, :H]
        sdx = a0[:, H:H + 3] + a1[:, H:H + 3]
        if l == 0:
            degv = a0[:, H + 3:H + 4] + a1[:, H + 3:H + 4] + 1.0
        else:
            degv = deg_r[...]
        xon = xo_r[...] + sdx / degv
        hv = h_r[...]
        upd = _silu(_dot(hv, wh1a_r[...]) + _dot(agg, wh1b_r[...]) + bh1_r[...])
        hn = hv + _dot(upd, wh2_r[...]) + bh2_r[...]
        ho_r[...] = hn
        xoo_r[...] = xon
        if l == 0:
            dego_r[...] = degv
        if not last:
            a = _dot(hn, wa_r[...])
            b = _dot(hn, wb_r[...])
            ccd = _dot(curv_r[...], wcd_r[...])
            ccs = _dot(curv_r[...], wcs_r[...])
            z = jnp.zeros((BN, TW - H - 4), jnp.float32)
            td_r[...] = jnp.concatenate([a, xon, ccd, z], axis=1)
            ts_r[...] = jnp.concatenate([b, -xon, ccs, z], axis=1)

    in_specs = [
        pl.BlockSpec((BN, H), lambda i: (i, 0)),
        pl.BlockSpec((NC, BN, TW), lambda i: (0, i, 0)),
        pl.BlockSpec((BN, 3), lambda i: (i, 0)),
    ]
    args = [h, acc, xo]
    if l != 0:
        in_specs.append(pl.BlockSpec((BN, 1), lambda i: (i, 0)))
        args.append(deg)
    if not last:
        in_specs += [pl.BlockSpec((BN, cd), lambda i: (i, 0)),
                     _full((H, H)), _full((H, H)), _full((cd, 1)),
                     _full((cd, 1))]
        args += [curvp, wa, wb, wcd, wcs]
    in_specs += [_full((H, H)), _full((H, H)), _full((1, H)), _full((H, H)),
                 _full((1, H))]
    args += [wh1a, wh1b, bh1, wh2, bh2]

    out_specs = [pl.BlockSpec((BN, H), lambda i: (i, 0)),
                 pl.BlockSpec((BN, 3), lambda i: (i, 0))]
    out_shape = [jax.ShapeDtypeStruct((n_pad, H), jnp.float32),
                 jax.ShapeDtypeStruct((n_pad, 3), jnp.float32)]
    if l == 0:
        out_specs.append(pl.BlockSpec((BN, 1), lambda i: (i, 0)))
        out_shape.append(jax.ShapeDtypeStruct((n_pad, 1), jnp.float32))
    if not last:
        out_specs += [pl.BlockSpec((BN, TW), lambda i: (i, 0)),
                      pl.BlockSpec((BN, TW), lambda i: (i, 0))]
        out_shape += [jax.ShapeDtypeStruct((n_pad, TW), jnp.float32),
                      jax.ShapeDtypeStruct((n_pad, TW), jnp.float32)]

    return pl.pallas_call(
        body, grid=(n_pad // BN,), in_specs=in_specs,
        out_specs=out_specs, out_shape=out_shape)(*args)


def _head(h, xo, batch2d, w_out, b_out, w_attn, b_attn, w_mu, b_mu, w_lv, b_lv,
          n_pad):
    aug = H + 1

    def body(h_r, xo_r, b_r, wout_r, bout_r, wattn_r, battn_r, wmu_r, bmu_r,
             wlv_r, blv_r, mu_r, lv_r, gmx_s):
        bt = b_r[...]
        gi = lax.broadcasted_iota(jnp.int32, (n_pad, G), 1)
        onehot = (bt == gi).astype(jnp.float32)
        valid = bt < G
        onesn = jnp.ones((n_pad, 1), jnp.float32)
        cdims = (((0,), (0,)), ((), ()))
        counts = jnp.maximum(
            lax.dot_general(onehot, onesn, cdims,
                            preferred_element_type=jnp.float32), 1.0)
        xov = xo_r[...]
        cent = lax.dot_general(onehot, xov, cdims,
                               preferred_element_type=jnp.float32) / counts
        cpn = _dot(onehot, cent)
        dv = xov - cpn
        dist = jnp.sqrt(jnp.sum(dv * dv, axis=1, keepdims=True))
        hout = _dot(h_r[...], wout_r[...]) + bout_r[...]
        haug = jnp.concatenate([hout, dist], axis=1)
        logits = _dot(haug, wattn_r[...]) + battn_r[0, 0]
        haug2 = jnp.concatenate([haug, logits], axis=1)

        def gmax_step(g, carry):
            mask = bt == g
            mx = jnp.max(jnp.where(mask, haug2, NEG), axis=0, keepdims=True)
            gmx_s[pl.ds(g, 1), :] = mx
            return carry

        lax.fori_loop(0, G, gmax_step, 0)
        gmx = gmx_s[...]
        gmax = gmx[:, :aug]
        lmax = gmx[:, aug:aug + 1]
        lpn = _dot(onehot, lmax)
        ex = jnp.where(valid, jnp.exp(logits - lpn), 0.0)
        denom = lax.dot_general(onehot, ex, cdims,
                                preferred_element_type=jnp.float32)
        dpn = _dot(onehot, denom)
        attn = jnp.where(valid, ex / dpn, 0.0)
        apool = lax.dot_general(onehot, attn * haug, cdims,
                                preferred_element_type=jnp.float32)
        gsum = lax.dot_general(onehot, haug, cdims,
                               preferred_element_type=jnp.float32)
        gmean = gsum / counts
        pooled = jnp.concatenate([gmean, gmax, apool], axis=1)
        mu_r[...] = _dot(pooled, wmu_r[...]) + bmu_r[...]
        lv_r[...] = jnp.clip(_dot(pooled, wlv_r[...]) + blv_r[...], -10.0, 10.0)

    lat = w_mu.shape[1]
    return pl.pallas_call(
        body,
        scratch_shapes=[pltpu.VMEM((G, aug + 1), jnp.float32)],
        out_shape=[jax.ShapeDtypeStruct((G, lat), jnp.float32),
                   jax.ShapeDtypeStruct((G, lat), jnp.float32)],
    )(h, xo, batch2d, w_out, b_out, w_attn, b_attn, w_mu, b_mu, w_lv, b_lv)


def kernel(x, pos, curv, edge_index, batch, W_in, b_in, We1, be1, We2, be2,
           Wc, bc, Wx, bx, Wh1, bh1, Wh2, bh2, W_out, b_out, W_attn, b_attn,
           W_mu, b_mu, W_lv, b_lv):
    n = x.shape[0]
    e = edge_index.shape[1]
    nl = We1.shape[0]
    cd = curv.shape[1]
    n_pad = _rup(n, NS * CHUNK)
    e_pad = _rup(e, NW * CHUNK)

    xp = jnp.pad(x, ((0, n_pad - n), (0, 0)))
    posp = jnp.pad(pos, ((0, n_pad - n), (0, 0)))
    curvp = jnp.pad(curv, ((0, n_pad - n), (0, 0)))
    batch2d = jnp.pad(batch, (0, n_pad - n),
                      constant_values=G).reshape(n_pad, 1)
    src = edge_index[0]
    dst = edge_index[1]
    dig = jnp.pad(dst, (0, e_pad - e))
    sig = jnp.pad(src, (0, e_pad - e))
    dsc = jnp.pad(dst, (0, e_pad - e), constant_values=n_pad - 1)
    ztile = jnp.zeros((CHUNK, TW), jnp.float32)

    h, td, ts = _entry(xp, posp, curvp, W_in, b_in.reshape(1, H),
                       We1[0, :H], We1[0, H:2 * H],
                       Wc[0, :cd], Wc[0, cd:], n_pad)
    xo = posp
    deg = None
    for l in range(nl):
        last = l == nl - 1
        gd, gs = _gather(td, ts, dig, sig, e_pad)
        s_arr = _edge(gd, gs, We2[l], Wx[l],
                      We1[l, 2 * H].reshape(1, H),
                      We1[l, 2 * H + 1].reshape(1, H),
                      be1[l].reshape(1, H), be2[l].reshape(1, H),
                      bc[l].reshape(1, 1), bx[l].reshape(1, 1), e_pad)
        acc = _scatter(s_arr, dsc, ztile, e_pad, n_pad)
        if last:
            nwa = nwb = nwcd = nwcs = None
        else:
            nwa, nwb = We1[l + 1, :H], We1[l + 1, H:2 * H]
            nwcd, nwcs = Wc[l + 1, :cd], Wc[l + 1, cd:]
        outs = _node(l, last, h, acc, xo, deg, curvp,
                     Wh1[l, :H], Wh1[l, H:], bh1[l].reshape(1, H),
                     Wh2[l], bh2[l].reshape(1, H),
                     nwa, nwb, nwcd, nwcs, n_pad)
        it = iter(outs)
        h = next(it)
        xo = next(it)
        if l == 0:
            deg = next(it)
        if not last:
            td = next(it)
            ts = next(it)

    mu, lv = _head(h, xo, batch2d, W_out, b_out.reshape(1, H), W_attn,
                   b_attn.reshape(1, 1), W_mu, b_mu.reshape(1, -1), W_lv,
                   b_lv.reshape(1, -1), n_pad)
    return (mu, lv)


# speedup vs baseline: 1.5192x; 1.5192x over previous
import jax
import jax.numpy as jnp
from jax import lax
from jax.experimental import pallas as pl
from jax.experimental.pallas import tpu as pltpu
from jax.experimental.pallas import tpu_sc as plsc

H = 128
G = 64
TW = 144
CHUNK = 128
NC, NS = 2, 16
NW = NC * NS
BN = 512
BE = 1024
NEG = -3.4e38


def _rup(a, b):
    return (a + b - 1) // b * b


def _silu(v):
    return v * jax.nn.sigmoid(v)


def _dot(a, b):
    return jnp.dot(a, b, preferred_element_type=jnp.float32)


def _dotx(a, b):
    return jnp.dot(a, b, preferred_element_type=jnp.float32,
                   precision=lax.Precision.HIGHEST)


def _bf(v):
    return v.astype(jnp.bfloat16).astype(jnp.float32)


def _full(shape):
    return pl.BlockSpec(shape, lambda i: (0,) * len(shape))


def _gather(td, dig, sig, e_pad):
    epw = e_pad // NW
    nch = epw // CHUNK
    mesh = plsc.VectorSubcoreMesh(core_axis_name="c", subcore_axis_name="s",
                                  num_cores=NC, num_subcores=NS)

    def body(td_h, di_h, si_h, gd_h, gs_h, di_v, si_v, rd_v, rs_v, s1, s2):
        wid = lax.axis_index("s") * NC + lax.axis_index("c")
        base = wid * epw

        def chunk(i, carry):
            off = base + i * CHUNK
            pltpu.sync_copy(di_h.at[pl.ds(off, CHUNK)], di_v)
            pltpu.sync_copy(si_h.at[pl.ds(off, CHUNK)], si_v)
            c1 = pltpu.async_copy(td_h.at[di_v], rd_v, s1)
            c2 = pltpu.async_copy(td_h.at[si_v], rs_v, s2)
            c1.wait()
            c2.wait()
            pltpu.sync_copy(rd_v, gd_h.at[pl.ds(off, CHUNK)])
            pltpu.sync_copy(rs_v, gs_h.at[pl.ds(off, CHUNK)])
            return carry

        lax.fori_loop(0, nch, chunk, 0)

    call = pl.kernel(
        body,
        out_type=(jax.ShapeDtypeStruct((e_pad, TW), jnp.float32),
                  jax.ShapeDtypeStruct((e_pad, TW), jnp.float32)),
        mesh=mesh,
        scratch_types=[
            pltpu.VMEM((CHUNK,), jnp.int32),
            pltpu.VMEM((CHUNK,), jnp.int32),
            pltpu.VMEM((CHUNK, TW), jnp.float32),
            pltpu.VMEM((CHUNK, TW), jnp.float32),
            pltpu.SemaphoreType.DMA,
            pltpu.SemaphoreType.DMA,
        ],
        compiler_params=pltpu.CompilerParams(use_tc_tiling_on_sc=False),
    )
    return call(td, dig, sig)


def _scatter(s_arr, dsc, ztile, e_pad, n_pad):
    epw = e_pad // NW
    nch = epw // CHUNK
    per_tile = n_pad // NS
    nzc = per_tile // CHUNK
    mesh = plsc.VectorSubcoreMesh(core_axis_name="c", subcore_axis_name="s",
                                  num_cores=NC, num_subcores=NS)

    def body(s_h, idx_h, z_h, out_h, idx_v, rows_v, acc_sh):
        c = lax.axis_index("c")
        s = lax.axis_index("s")
        wid = s * NC + c
        pltpu.sync_copy(z_h, rows_v)
        for k in range(nzc):
            pltpu.sync_copy(rows_v,
                            acc_sh.at[pl.ds(s * per_tile + k * CHUNK, CHUNK)])
        plsc.subcore_barrier()

        def chunk(i, carry):
            off = wid * epw + i * CHUNK
            pltpu.sync_copy(idx_h.at[pl.ds(off, CHUNK)], idx_v)
            pltpu.sync_copy(s_h.at[pl.ds(off, CHUNK)], rows_v)
            pltpu.sync_copy(rows_v, acc_sh.at[idx_v], add=True)
            return carry

        lax.fori_loop(0, nch, chunk, 0)
        plsc.subcore_barrier()
        for k in range(nzc):
            rof = s * per_tile + k * CHUNK
            pltpu.sync_copy(acc_sh.at[pl.ds(rof, CHUNK)], rows_v)
            pltpu.sync_copy(rows_v, out_h.at[c, pl.ds(rof, CHUNK)])

    call = pl.kernel(
        body,
        out_type=jax.ShapeDtypeStruct((NC, n_pad, TW), jnp.float32),
        mesh=mesh,
        scratch_types=[
            pltpu.VMEM((CHUNK,), jnp.int32),
            pltpu.VMEM((CHUNK, TW), jnp.float32),
            pltpu.VMEM_SHARED((n_pad, TW), jnp.float32),
        ],
        compiler_params=pltpu.CompilerParams(use_tc_tiling_on_sc=False),
    )
    return call(s_arr, dsc, ztile)


def _entry(xp, posp, curvp, w_in, b_in, n_pad):
    nin = xp.shape[1]
    cd = curvp.shape[1]

    def body(x_r, pos_r, curv_r, win_r, bin_r, h_r, td_r):
        h = _silu(_dot(x_r[...], win_r[...]) + bin_r[...])
        z = jnp.zeros((BN, TW - H - 3 - cd), jnp.float32)
        h_r[...] = h
        td_r[...] = jnp.concatenate([h, pos_r[...], curv_r[...], z], axis=1)

    return pl.pallas_call(
        body,
        grid=(n_pad // BN,),
        in_specs=[
            pl.BlockSpec((BN, nin), lambda i: (i, 0)),
            pl.BlockSpec((BN, 3), lambda i: (i, 0)),
            pl.BlockSpec((BN, cd), lambda i: (i, 0)),
            _full((nin, H)), _full((1, H)),
        ],
        out_specs=[
            pl.BlockSpec((BN, H), lambda i: (i, 0)),
            pl.BlockSpec((BN, TW), lambda i: (i, 0)),
        ],
        out_shape=[
            jax.ShapeDtypeStruct((n_pad, H), jnp.float32),
            jax.ShapeDtypeStruct((n_pad, TW), jnp.float32),
        ],
    )(xp, posp, curvp, w_in, b_in)


def _edge(gd, gs, we1, wc, we2, wx, be1, be2, bc, bx, cd, e_pad):
    k1 = 2 * H + 2

    def body(gd_r, gs_r, we1_r, wc_r, we2_r, wx_r, be1_r, be2_r, bc_r, bx_r,
             out_r):
        gdv = gd_r[...]
        gsv = gs_r[...]
        hi = gdv[:, :H]
        hj = gsv[:, :H]
        diff = gdv[:, H:H + 3] - gsv[:, H:H + 3]
        d2 = jnp.sum(diff * diff, axis=1, keepdims=True)
        cwcat = jnp.concatenate([gdv[:, H + 3:H + 3 + cd],
                                 gsv[:, H + 3:H + 3 + cd]], axis=1)
        cw = jax.nn.sigmoid(_dot(cwcat, wc_r[...]) + bc_r[0, 0])
        ebig = jnp.concatenate([hi, hj, d2, cw], axis=1)
        m1 = _silu(_dot(ebig, we1_r[...]) + be1_r[...])
        m2 = _silu(_dot(m1, we2_r[...]) + be2_r[...]) * cw
        coef = jnp.tanh(_dot(m2, wx_r[...]) + bx_r[0, 0])
        dx = diff * coef
        ones = jnp.ones((BE, 1), jnp.float32)
        z = jnp.zeros((BE, TW - H - 4), jnp.float32)
        out_r[...] = jnp.concatenate([m2, dx, ones, z], axis=1)

    return pl.pallas_call(
        body,
        grid=(e_pad // BE,),
        in_specs=[
            pl.BlockSpec((BE, TW), lambda i: (i, 0)),
            pl.BlockSpec((BE, TW), lambda i: (i, 0)),
            _full((k1, H)), _full((2 * cd, 1)), _full((H, H)), _full((H, 1)),
            _full((1, H)), _full((1, H)), _full((1, 1)), _full((1, 1)),
        ],
        out_specs=pl.BlockSpec((BE, TW), lambda i: (i, 0)),
        out_shape=jax.ShapeDtypeStruct((e_pad, TW), jnp.float32),
    )(gd, gs, we1, wc, we2, wx, be1, be2, bc, bx)


def _node(l, last, h, acc, xo, deg, curvp, wh1, bh1, wh2, bh2, n_pad):
    cd = curvp.shape[1]

    def body(*refs):
        it = iter(refs)
        h_r = next(it)
        acc_r = next(it)
        xo_r = next(it)
        deg_r = None if l == 0 else next(it)
        curv_r = next(it) if not last else None
        wh1_r = next(it)
        bh1_r = next(it)
        wh2_r = next(it)
        bh2_r = next(it)
        ho_r = next(it)
        xoo_r = next(it)
        dego_r = next(it) if l == 0 else None
        td_r = next(it) if not last else None

        a0 = acc_r[0]
        a1 = acc_r[1]
        agg = a0[:, :H] + a1[:, :H]
        sdx = a0[:, H:H + 3] + a1[:, H:H + 3]
        if l == 0:
            degv = a0[:, H + 3:H + 4] + a1[:, H + 3:H + 4] + 1.0
        else:
            degv = deg_r[...]
        xon = xo_r[...] + sdx / degv
        hv = h_r[...]
        hcat = jnp.concatenate([hv, agg], axis=1)
        upd = _silu(_dot(hcat, wh1_r[...]) + bh1_r[...])
        hn = hv + _dot(upd, wh2_r[...]) + bh2_r[...]
        ho_r[...] = hn
        xoo_r[...] = xon
        if l == 0:
            dego_r[...] = degv
        if not last:
            z = jnp.zeros((BN, TW - H - 3 - cd), jnp.float32)
            td_r[...] = jnp.concatenate([hn, xon, curv_r[...], z], axis=1)

    in_specs = [
        pl.BlockSpec((BN, H), lambda i: (i, 0)),
        pl.BlockSpec((NC, BN, TW), lambda i: (0, i, 0)),
        pl.BlockSpec((BN, 3), lambda i: (i, 0)),
    ]
    args = [h, acc, xo]
    if l != 0:
        in_specs.append(pl.BlockSpec((BN, 1), lambda i: (i, 0)))
        args.append(deg)
    if not last:
        in_specs.append(pl.BlockSpec((BN, cd), lambda i: (i, 0)))
        args.append(curvp)
    in_specs += [_full((2 * H, H)), _full((1, H)), _full((H, H)),
                 _full((1, H))]
    args += [wh1, bh1, wh2, bh2]

    out_specs = [pl.BlockSpec((BN, H), lambda i: (i, 0)),
                 pl.BlockSpec((BN, 3), lambda i: (i, 0))]
    out_shape = [jax.ShapeDtypeStruct((n_pad, H), jnp.float32),
                 jax.ShapeDtypeStruct((n_pad, 3), jnp.float32)]
    if l == 0:
        out_specs.append(pl.BlockSpec((BN, 1), lambda i: (i, 0)))
        out_shape.append(jax.ShapeDtypeStruct((n_pad, 1), jnp.float32))
    if not last:
        out_specs.append(pl.BlockSpec((BN, TW), lambda i: (i, 0)))
        out_shape.append(jax.ShapeDtypeStruct((n_pad, TW), jnp.float32))

    return pl.pallas_call(
        body, grid=(n_pad // BN,), in_specs=in_specs,
        out_specs=out_specs, out_shape=out_shape)(*args)


def _head(h, xo, batch2d, w_out, b_out, w_attn, b_attn, w_mu, b_mu, w_lv, b_lv,
          n_pad):
    aug = H + 1
    nblk = n_pad // BN
    cdims = (((0,), (0,)), ((), ()))

    def _seg(oh, v):
        return lax.dot_general(oh, v, cdims,
                               preferred_element_type=jnp.float32,
                               precision=lax.Precision.HIGHEST)

    def body(h_r, xo_r, b_r, wout_r, bout_r, wattn_r, battn_r, wmu_r, bmu_r,
             wlv_r, blv_r, mu_r, lv_r, cnt_s, xsum_s, gsum_s, gmx_s, den_s,
             ap_s):
        s = pl.program_id(0)
        i = pl.program_id(1)
        bt = b_r[...]
        gi = lax.broadcasted_iota(jnp.int32, (BN, G), 1)
        oh = (bt == gi).astype(jnp.float32)
        valid = bt < G
        xov = xo_r[...]

        @pl.when(jnp.logical_and(s == 0, i == 0))
        def _init():
            cnt_s[...] = jnp.zeros((G, 1), jnp.float32)
            xsum_s[...] = jnp.zeros((G, 3), jnp.float32)
            gsum_s[...] = jnp.zeros((G, aug), jnp.float32)
            gmx_s[...] = jnp.full((G, aug + 1), NEG, jnp.float32)
            den_s[...] = jnp.zeros((G, 1), jnp.float32)
            ap_s[...] = jnp.zeros((G, aug), jnp.float32)

        @pl.when(s == 0)
        def _s0():
            cnt_s[...] += _seg(oh, jnp.ones((BN, 1), jnp.float32))
            xsum_s[...] += _seg(oh, xov)

        def _haug_logits():
            counts = jnp.maximum(cnt_s[...], 1.0)
            cent = xsum_s[...] / counts
            cpn = _dotx(oh, cent)
            dv = xov - cpn
            dist = jnp.sqrt(jnp.sum(dv * dv, axis=1, keepdims=True))
            hout = _dot(h_r[...], wout_r[...]) + bout_r[...]
            haug = jnp.concatenate([hout, dist], axis=1)
            logits = _dot(haug, wattn_r[...]) + battn_r[0, 0]
            return haug, logits

        @pl.when(s == 1)
        def _s1():
            haug, logits = _haug_logits()
            gsum_s[...] += _seg(oh, haug)
            haug2 = jnp.concatenate([haug, logits], axis=1)

            def gmax_step(g, carry):
                mask = bt == g
                mx = jnp.max(jnp.where(mask, haug2, NEG), axis=0,
                             keepdims=True)
                gmx_s[pl.ds(g, 1), :] = jnp.maximum(gmx_s[pl.ds(g, 1), :], mx)
                return carry

            lax.fori_loop(0, G, gmax_step, 0)

        @pl.when(s == 2)
        def _s2():
            _, logits = _haug_logits()
            lpn = _dotx(oh, gmx_s[:, aug:aug + 1])
            ex = jnp.where(valid, jnp.exp(logits - lpn), 0.0)
            den_s[...] += _seg(oh, ex)

        @pl.when(s == 3)
        def _s3():
            haug, logits = _haug_logits()
            lpn = _dotx(oh, gmx_s[:, aug:aug + 1])
            ex = jnp.where(valid, jnp.exp(logits - lpn), 0.0)
            dpn = _dotx(oh, den_s[...])
            attn = jnp.where(valid, ex / dpn, 0.0)
            ap_s[...] += _seg(oh, attn * haug)

        @pl.when(jnp.logical_and(s == 3, i == nblk - 1))
        def _fin():
            counts = jnp.maximum(cnt_s[...], 1.0)
            gmean = gsum_s[...] / counts
            pooled = jnp.concatenate([gmean, gmx_s[:, :aug], ap_s[...]],
                                     axis=1)
            mu_r[...] = _dot(pooled, wmu_r[...]) + bmu_r[...]
            lv_r[...] = jnp.clip(_dot(pooled, wlv_r[...]) + blv_r[...],
                                 -10.0, 10.0)

    lat = w_mu.shape[1]
    blk = lambda w: pl.BlockSpec((BN, w), lambda s, i: (i, 0))
    fullg = lambda shp: pl.BlockSpec(shp, lambda s, i: (0,) * len(shp))
    return pl.pallas_call(
        body,
        grid=(4, nblk),
        in_specs=[
            blk(H), blk(3), blk(1),
            fullg((H, H)), fullg((1, H)), fullg((aug, 1)), fullg((1, 1)),
            fullg((3 * aug, lat)), fullg((1, lat)), fullg((3 * aug, lat)),
            fullg((1, lat)),
        ],
        out_specs=[pl.BlockSpec((G, lat), lambda s, i: (0, 0)),
                   pl.BlockSpec((G, lat), lambda s, i: (0, 0))],
        scratch_shapes=[
            pltpu.VMEM((G, 1), jnp.float32),
            pltpu.VMEM((G, 3), jnp.float32),
            pltpu.VMEM((G, aug), jnp.float32),
            pltpu.VMEM((G, aug + 1), jnp.float32),
            pltpu.VMEM((G, 1), jnp.float32),
            pltpu.VMEM((G, aug), jnp.float32),
        ],
        out_shape=[jax.ShapeDtypeStruct((G, lat), jnp.float32),
                   jax.ShapeDtypeStruct((G, lat), jnp.float32)],
    )(h, xo, batch2d, w_out, b_out, w_attn, b_attn, w_mu, b_mu, w_lv, b_lv)


def kernel(x, pos, curv, edge_index, batch, W_in, b_in, We1, be1, We2, be2,
           Wc, bc, Wx, bx, Wh1, bh1, Wh2, bh2, W_out, b_out, W_attn, b_attn,
           W_mu, b_mu, W_lv, b_lv):
    n = x.shape[0]
    e = edge_index.shape[1]
    nl = We1.shape[0]
    cd = curv.shape[1]
    n_pad = _rup(n, NS * CHUNK)
    e_pad = _rup(e, NW * CHUNK)

    xp = jnp.pad(x, ((0, n_pad - n), (0, 0)))
    posp = jnp.pad(pos, ((0, n_pad - n), (0, 0)))
    curvp = jnp.pad(curv, ((0, n_pad - n), (0, 0)))
    batch2d = jnp.pad(batch, (0, n_pad - n),
                      constant_values=G).reshape(n_pad, 1)
    src = edge_index[0]
    dst = edge_index[1]
    dig = jnp.pad(dst, (0, e_pad - e))
    sig = jnp.pad(src, (0, e_pad - e))
    dsc = jnp.pad(dst, (0, e_pad - e), constant_values=n_pad - 1)
    ztile = jnp.zeros((CHUNK, TW), jnp.float32)

    h, td = _entry(xp, posp, curvp, W_in, b_in.reshape(1, H), n_pad)
    xo = posp
    deg = None
    for l in range(nl):
        last = l == nl - 1
        gd, gs = _gather(td, dig, sig, e_pad)
        s_arr = _edge(gd, gs, We1[l], Wc[l], We2[l], Wx[l],
                      be1[l].reshape(1, H), be2[l].reshape(1, H),
                      bc[l].reshape(1, 1), bx[l].reshape(1, 1), cd, e_pad)
        acc = _scatter(s_arr, dsc, ztile, e_pad, n_pad)
        outs = _node(l, last, h, acc, xo, deg, curvp,
                     Wh1[l], bh1[l].reshape(1, H),
                     Wh2[l], bh2[l].reshape(1, H), n_pad)
        it = iter(outs)
        h = next(it)
        xo = next(it)
        if l == 0:
            deg = next(it)
        if not last:
            td = next(it)

    mu, lv = _head(h, xo, batch2d, W_out, b_out.reshape(1, H), W_attn,
                   b_attn.reshape(1, 1), W_mu, b_mu.reshape(1, -1), W_lv,
                   b_lv.reshape(1, -1), n_pad)
    return (mu, lv)


# speedup vs baseline: 1.5890x; 1.0460x over previous
import jax
import jax.numpy as jnp
from jax import lax
from jax.experimental import pallas as pl
from jax.experimental.pallas import tpu as pltpu
from jax.experimental.pallas import tpu_sc as plsc

H = 128
G = 64
TW = 144
CHUNK = 128
NC, NS = 2, 16
NW = NC * NS
BN = 512
BE = 1024
NEG = -3.4e38


def _rup(a, b):
    return (a + b - 1) // b * b


def _silu(v):
    return v * jax.nn.sigmoid(v)


def _dot(a, b):
    return jnp.dot(a, b, preferred_element_type=jnp.float32)


def _dotx(a, b):
    return jnp.dot(a, b, preferred_element_type=jnp.float32,
                   precision=lax.Precision.HIGHEST)


def _bf(v):
    return v.astype(jnp.bfloat16).astype(jnp.float32)


def _full(shape):
    return pl.BlockSpec(shape, lambda i: (0,) * len(shape))


def _gather(td, di2, si2, e_pad):
    epw = e_pad // NW
    nch = epw // CHUNK
    GRP = 2
    SUB = GRP * CHUNK
    ngr = epw // SUB
    mesh = plsc.VectorSubcoreMesh(core_axis_name="c", subcore_axis_name="s",
                                  num_cores=NC, num_subcores=NS)

    def body(td_h, di_h, si_h, gd_h, gs_h, di_v, si_v, rd_v, rs_v, gsem):
        wid = lax.axis_index("s") * NC + lax.axis_index("c")
        rbase = wid * nch
        ebase = wid * epw
        pltpu.sync_copy(di_h.at[pl.ds(rbase, nch)], di_v)
        pltpu.sync_copy(si_h.at[pl.ds(rbase, nch)], si_v)

        def grp(g, carry):
            cs = []
            for b in range(GRP):
                i = g * GRP + b
                cs.append(pltpu.async_copy(
                    td_h.at[di_v.at[i]],
                    rd_v.at[pl.ds(b * CHUNK, CHUNK)], gsem))
                cs.append(pltpu.async_copy(
                    td_h.at[si_v.at[i]],
                    rs_v.at[pl.ds(b * CHUNK, CHUNK)], gsem))
            for c in cs:
                c.wait()
            off = ebase + g * SUB
            pltpu.sync_copy(rd_v, gd_h.at[pl.ds(off, SUB)])
            pltpu.sync_copy(rs_v, gs_h.at[pl.ds(off, SUB)])
            return carry

        lax.fori_loop(0, ngr, grp, 0)

    call = pl.kernel(
        body,
        out_type=(jax.ShapeDtypeStruct((e_pad, TW), jnp.float32),
                  jax.ShapeDtypeStruct((e_pad, TW), jnp.float32)),
        mesh=mesh,
        scratch_types=[
            pltpu.VMEM((nch, CHUNK), jnp.int32),
            pltpu.VMEM((nch, CHUNK), jnp.int32),
            pltpu.VMEM((SUB, TW), jnp.float32),
            pltpu.VMEM((SUB, TW), jnp.float32),
            pltpu.SemaphoreType.DMA,
        ],
        compiler_params=pltpu.CompilerParams(use_tc_tiling_on_sc=False),
    )
    return call(td, di2, si2)


def _scatter(s_arr, dsc2, ztile, e_pad, n_pad):
    epw = e_pad // NW
    nch = epw // CHUNK
    GRP = 2
    SUB = GRP * CHUNK
    ngr = epw // SUB
    per_tile = n_pad // NS
    nzc = per_tile // CHUNK
    mesh = plsc.VectorSubcoreMesh(core_axis_name="c", subcore_axis_name="s",
                                  num_cores=NC, num_subcores=NS)

    def body(s_h, idx_h, z_h, out_h, idx_v, rows_v, acc_sh, lsem):
        c = lax.axis_index("c")
        s = lax.axis_index("s")
        wid = s * NC + c
        rbase = wid * nch
        ebase = wid * epw
        zb = rows_v.at[pl.ds(0, CHUNK)]
        pltpu.sync_copy(z_h, zb)
        for k in range(nzc):
            pltpu.sync_copy(zb,
                            acc_sh.at[pl.ds(s * per_tile + k * CHUNK, CHUNK)])
        plsc.subcore_barrier()

        def grp(g, carry):
            off = ebase + g * SUB
            pltpu.sync_copy(idx_h.at[pl.ds(rbase + g * GRP, GRP)], idx_v)
            pltpu.async_copy(s_h.at[pl.ds(off, SUB)], rows_v, lsem).wait()
            for b in range(GRP):
                pltpu.sync_copy(rows_v.at[pl.ds(b * CHUNK, CHUNK)],
                                acc_sh.at[idx_v.at[b]], add=True)
            return carry

        lax.fori_loop(0, ngr, grp, 0)
        plsc.subcore_barrier()
        wb = rows_v.at[pl.ds(0, CHUNK)]
        for k in range(nzc):
            rof = s * per_tile + k * CHUNK
            pltpu.sync_copy(acc_sh.at[pl.ds(rof, CHUNK)], wb)
            pltpu.sync_copy(wb, out_h.at[c, pl.ds(rof, CHUNK)])

    call = pl.kernel(
        body,
        out_type=jax.ShapeDtypeStruct((NC, n_pad, TW), jnp.float32),
        mesh=mesh,
        scratch_types=[
            pltpu.VMEM((GRP, CHUNK), jnp.int32),
            pltpu.VMEM((SUB, TW), jnp.float32),
            pltpu.VMEM_SHARED((n_pad, TW), jnp.float32),
            pltpu.SemaphoreType.DMA,
        ],
        compiler_params=pltpu.CompilerParams(use_tc_tiling_on_sc=False),
    )
    return call(s_arr, dsc2, ztile)


def _entry(xp, posp, curvp, w_in, b_in, n_pad):
    nin = xp.shape[1]
    cd = curvp.shape[1]

    def body(x_r, pos_r, curv_r, win_r, bin_r, h_r, td_r):
        h = _silu(_dot(x_r[...], win_r[...]) + bin_r[...])
        z = jnp.zeros((BN, TW - H - 3 - cd), jnp.float32)
        h_r[...] = h
        td_r[...] = jnp.concatenate([h, pos_r[...], curv_r[...], z], axis=1)

    return pl.pallas_call(
        body,
        grid=(n_pad // BN,),
        in_specs=[
            pl.BlockSpec((BN, nin), lambda i: (i, 0)),
            pl.BlockSpec((BN, 3), lambda i: (i, 0)),
            pl.BlockSpec((BN, cd), lambda i: (i, 0)),
            _full((nin, H)), _full((1, H)),
        ],
        out_specs=[
            pl.BlockSpec((BN, H), lambda i: (i, 0)),
            pl.BlockSpec((BN, TW), lambda i: (i, 0)),
        ],
        out_shape=[
            jax.ShapeDtypeStruct((n_pad, H), jnp.float32),
            jax.ShapeDtypeStruct((n_pad, TW), jnp.float32),
        ],
    )(xp, posp, curvp, w_in, b_in)


def _edge(gd, gs, we1, wc, we2, wx, be1, be2, bc, bx, cd, e_pad):
    k1 = 2 * H + 2

    def body(gd_r, gs_r, we1_r, wc_r, we2_r, wx_r, be1_r, be2_r, bc_r, bx_r,
             out_r):
        gdv = gd_r[...]
        gsv = gs_r[...]
        hi = gdv[:, :H]
        hj = gsv[:, :H]
        diff = gdv[:, H:H + 3] - gsv[:, H:H + 3]
        d2 = jnp.sum(diff * diff, axis=1, keepdims=True)
        cwcat = jnp.concatenate([gdv[:, H + 3:H + 3 + cd],
                                 gsv[:, H + 3:H + 3 + cd]], axis=1)
        cw = jax.nn.sigmoid(_dot(cwcat, wc_r[...]) + bc_r[0, 0])
        ebig = jnp.concatenate([hi, hj, d2, cw], axis=1)
        m1 = _silu(_dot(ebig, we1_r[...]) + be1_r[...])
        m2 = _silu(_dot(m1, we2_r[...]) + be2_r[...]) * cw
        coef = jnp.tanh(_dot(m2, wx_r[...]) + bx_r[0, 0])
        dx = diff * coef
        ones = jnp.ones((BE, 1), jnp.float32)
        z = jnp.zeros((BE, TW - H - 4), jnp.float32)
        out_r[...] = jnp.concatenate([m2, dx, ones, z], axis=1)

    return pl.pallas_call(
        body,
        grid=(e_pad // BE,),
        in_specs=[
            pl.BlockSpec((BE, TW), lambda i: (i, 0)),
            pl.BlockSpec((BE, TW), lambda i: (i, 0)),
            _full((k1, H)), _full((2 * cd, 1)), _full((H, H)), _full((H, 1)),
            _full((1, H)), _full((1, H)), _full((1, 1)), _full((1, 1)),
        ],
        out_specs=pl.BlockSpec((BE, TW), lambda i: (i, 0)),
        out_shape=jax.ShapeDtypeStruct((e_pad, TW), jnp.float32),
    )(gd, gs, we1, wc, we2, wx, be1, be2, bc, bx)


def _node(l, last, h, acc, xo, deg, curvp, wh1, bh1, wh2, bh2, n_pad):
    cd = curvp.shape[1]

    def body(*refs):
        it = iter(refs)
        h_r = next(it)
        acc_r = next(it)
        xo_r = next(it)
        deg_r = None if l == 0 else next(it)
        curv_r = next(it) if not last else None
        wh1_r = next(it)
        bh1_r = next(it)
        wh2_r = next(it)
        bh2_r = next(it)
        ho_r = next(it)
        xoo_r = next(it)
        dego_r = next(it) if l == 0 else None
        td_r = next(it) if not last else None

        a0 = acc_r[0]
        a1 = acc_r[1]
        agg = a0[:, :H] + a1[:, :H]
        sdx = a0[:, H:H + 3] + a1[:, H:H + 3]
        if l == 0:
            degv = a0[:, H + 3:H + 4] + a1[:, H + 3:H + 4] + 1.0
        else:
            degv = deg_r[...]
        xon = xo_r[...] + sdx / degv
        hv = h_r[...]
        hcat = jnp.concatenate([hv, agg], axis=1)
        upd = _silu(_dot(hcat, wh1_r[...]) + bh1_r[...])
        hn = hv + _dot(upd, wh2_r[...]) + bh2_r[...]
        ho_r[...] = hn
        xoo_r[...] = xon
        if l == 0:
            dego_r[...] = degv
        if not last:
            z = jnp.zeros((BN, TW - H - 3 - cd), jnp.float32)
            td_r[...] = jnp.concatenate([hn, xon, curv_r[...], z], axis=1)

    in_specs = [
        pl.BlockSpec((BN, H), lambda i: (i, 0)),
        pl.BlockSpec((NC, BN, TW), lambda i: (0, i, 0)),
        pl.BlockSpec((BN, 3), lambda i: (i, 0)),
    ]
    args = [h, acc, xo]
    if l != 0:
        in_specs.append(pl.BlockSpec((BN, 1), lambda i: (i, 0)))
        args.append(deg)
    if not last:
        in_specs.append(pl.BlockSpec((BN, cd), lambda i: (i, 0)))
        args.append(curvp)
    in_specs += [_full((2 * H, H)), _full((1, H)), _full((H, H)),
                 _full((1, H))]
    args += [wh1, bh1, wh2, bh2]

    out_specs = [pl.BlockSpec((BN, H), lambda i: (i, 0)),
                 pl.BlockSpec((BN, 3), lambda i: (i, 0))]
    out_shape = [jax.ShapeDtypeStruct((n_pad, H), jnp.float32),
                 jax.ShapeDtypeStruct((n_pad, 3), jnp.float32)]
    if l == 0:
        out_specs.append(pl.BlockSpec((BN, 1), lambda i: (i, 0)))
        out_shape.append(jax.ShapeDtypeStruct((n_pad, 1), jnp.float32))
    if not last:
        out_specs.append(pl.BlockSpec((BN, TW), lambda i: (i, 0)))
        out_shape.append(jax.ShapeDtypeStruct((n_pad, TW), jnp.float32))

    return pl.pallas_call(
        body, grid=(n_pad // BN,), in_specs=in_specs,
        out_specs=out_specs, out_shape=out_shape)(*args)


def _head(h, xo, batch2d, w_out, b_out, w_attn, b_attn, w_mu, b_mu, w_lv, b_lv,
          n_pad):
    aug = H + 1
    nblk = n_pad // BN
    cdims = (((0,), (0,)), ((), ()))

    def _seg(oh, v):
        return lax.dot_general(oh, v, cdims,
                               preferred_element_type=jnp.float32,
                               precision=lax.Precision.HIGHEST)

    def body(h_r, xo_r, b_r, wout_r, bout_r, wattn_r, battn_r, wmu_r, bmu_r,
             wlv_r, blv_r, mu_r, lv_r, cnt_s, xsum_s, gsum_s, gmx_s, den_s,
             ap_s):
        s = pl.program_id(0)
        i = pl.program_id(1)
        bt = b_r[...]
        gi = lax.broadcasted_iota(jnp.int32, (BN, G), 1)
        oh = (bt == gi).astype(jnp.float32)
        valid = bt < G
        xov = xo_r[...]

        @pl.when(jnp.logical_and(s == 0, i == 0))
        def _init():
            cnt_s[...] = jnp.zeros((G, 1), jnp.float32)
            xsum_s[...] = jnp.zeros((G, 3), jnp.float32)
            gsum_s[...] = jnp.zeros((G, aug), jnp.float32)
            gmx_s[...] = jnp.full((G, aug + 1), NEG, jnp.float32)
            den_s[...] = jnp.zeros((G, 1), jnp.float32)
            ap_s[...] = jnp.zeros((G, aug), jnp.float32)

        @pl.when(s == 0)
        def _s0():
            cnt_s[...] += _seg(oh, jnp.ones((BN, 1), jnp.float32))
            xsum_s[...] += _seg(oh, xov)

        def _haug_logits():
            counts = jnp.maximum(cnt_s[...], 1.0)
            cent = xsum_s[...] / counts
            cpn = _dotx(oh, cent)
            dv = xov - cpn
            dist = jnp.sqrt(jnp.sum(dv * dv, axis=1, keepdims=True))
            hout = _dot(h_r[...], wout_r[...]) + bout_r[...]
            haug = jnp.concatenate([hout, dist], axis=1)
            logits = _dot(haug, wattn_r[...]) + battn_r[0, 0]
            return haug, logits

        @pl.when(s == 1)
        def _s1():
            haug, logits = _haug_logits()
            gsum_s[...] += _seg(oh, haug)
            haug2 = jnp.concatenate([haug, logits], axis=1)

            def gmax_step(g, carry):
                mask = bt == g
                mx = jnp.max(jnp.where(mask, haug2, NEG), axis=0,
                             keepdims=True)
                gmx_s[pl.ds(g, 1), :] = jnp.maximum(gmx_s[pl.ds(g, 1), :], mx)
                return carry

            lax.fori_loop(0, G, gmax_step, 0)

        @pl.when(s == 2)
        def _s2():
            _, logits = _haug_logits()
            lpn = _dotx(oh, gmx_s[:, aug:aug + 1])
            ex = jnp.where(valid, jnp.exp(logits - lpn), 0.0)
            den_s[...] += _seg(oh, ex)

        @pl.when(s == 3)
        def _s3():
            haug, logits = _haug_logits()
            lpn = _dotx(oh, gmx_s[:, aug:aug + 1])
            ex = jnp.where(valid, jnp.exp(logits - lpn), 0.0)
            dpn = _dotx(oh, den_s[...])
            attn = jnp.where(valid, ex / dpn, 0.0)
            ap_s[...] += _seg(oh, attn * haug)

        @pl.when(jnp.logical_and(s == 3, i == nblk - 1))
        def _fin():
            counts = jnp.maximum(cnt_s[...], 1.0)
            gmean = gsum_s[...] / counts
            pooled = jnp.concatenate([gmean, gmx_s[:, :aug], ap_s[...]],
                                     axis=1)
            mu_r[...] = _dot(pooled, wmu_r[...]) + bmu_r[...]
            lv_r[...] = jnp.clip(_dot(pooled, wlv_r[...]) + blv_r[...],
                                 -10.0, 10.0)

    lat = w_mu.shape[1]
    blk = lambda w: pl.BlockSpec((BN, w), lambda s, i: (i, 0))
    fullg = lambda shp: pl.BlockSpec(shp, lambda s, i: (0,) * len(shp))
    return pl.pallas_call(
        body,
        grid=(4, nblk),
        in_specs=[
            blk(H), blk(3), blk(1),
            fullg((H, H)), fullg((1, H)), fullg((aug, 1)), fullg((1, 1)),
            fullg((3 * aug, lat)), fullg((1, lat)), fullg((3 * aug, lat)),
            fullg((1, lat)),
        ],
        out_specs=[pl.BlockSpec((G, lat), lambda s, i: (0, 0)),
                   pl.BlockSpec((G, lat), lambda s, i: (0, 0))],
        scratch_shapes=[
            pltpu.VMEM((G, 1), jnp.float32),
            pltpu.VMEM((G, 3), jnp.float32),
            pltpu.VMEM((G, aug), jnp.float32),
            pltpu.VMEM((G, aug + 1), jnp.float32),
            pltpu.VMEM((G, 1), jnp.float32),
            pltpu.VMEM((G, aug), jnp.float32),
        ],
        out_shape=[jax.ShapeDtypeStruct((G, lat), jnp.float32),
                   jax.ShapeDtypeStruct((G, lat), jnp.float32)],
    )(h, xo, batch2d, w_out, b_out, w_attn, b_attn, w_mu, b_mu, w_lv, b_lv)


def kernel(x, pos, curv, edge_index, batch, W_in, b_in, We1, be1, We2, be2,
           Wc, bc, Wx, bx, Wh1, bh1, Wh2, bh2, W_out, b_out, W_attn, b_attn,
           W_mu, b_mu, W_lv, b_lv):
    n = x.shape[0]
    e = edge_index.shape[1]
    nl = We1.shape[0]
    cd = curv.shape[1]
    n_pad = _rup(n, NS * CHUNK)
    e_pad = _rup(e, NW * CHUNK)

    xp = jnp.pad(x, ((0, n_pad - n), (0, 0)))
    posp = jnp.pad(pos, ((0, n_pad - n), (0, 0)))
    curvp = jnp.pad(curv, ((0, n_pad - n), (0, 0)))
    batch2d = jnp.pad(batch, (0, n_pad - n),
                      constant_values=G).reshape(n_pad, 1)
    src = edge_index[0]
    dst = edge_index[1]
    dig = jnp.pad(dst, (0, e_pad - e)).reshape(e_pad // CHUNK, CHUNK)
    sig = jnp.pad(src, (0, e_pad - e)).reshape(e_pad // CHUNK, CHUNK)
    dsc = jnp.pad(dst, (0, e_pad - e),
                  constant_values=n_pad - 1).reshape(e_pad // CHUNK, CHUNK)
    ztile = jnp.zeros((CHUNK, TW), jnp.float32)

    h, td = _entry(xp, posp, curvp, W_in, b_in.reshape(1, H), n_pad)
    xo = posp
    deg = None
    for l in range(nl):
        last = l == nl - 1
        gd, gs = _gather(td, dig, sig, e_pad)
        s_arr = _edge(gd, gs, We1[l], Wc[l], We2[l], Wx[l],
                      be1[l].reshape(1, H), be2[l].reshape(1, H),
                      bc[l].reshape(1, 1), bx[l].reshape(1, 1), cd, e_pad)
        acc = _scatter(s_arr, dsc, ztile, e_pad, n_pad)
        outs = _node(l, last, h, acc, xo, deg, curvp,
                     Wh1[l], bh1[l].reshape(1, H),
                     Wh2[l], bh2[l].reshape(1, H), n_pad)
        it = iter(outs)
        h = next(it)
        xo = next(it)
        if l == 0:
            deg = next(it)
        if not last:
            td = next(it)

    mu, lv = _head(h, xo, batch2d, W_out, b_out.reshape(1, H), W_attn,
                   b_attn.reshape(1, 1), W_mu, b_mu.reshape(1, -1), W_lv,
                   b_lv.reshape(1, -1), n_pad)
    return (mu, lv)


# speedup vs baseline: 1.6343x; 1.0285x over previous
import jax
import jax.numpy as jnp
from jax import lax
from jax.experimental import pallas as pl
from jax.experimental.pallas import tpu as pltpu
from jax.experimental.pallas import tpu_sc as plsc

H = 128
G = 64
TW = 144
CHUNK = 128
NC, NS = 2, 16
NW = NC * NS
BN = 512
BE = 1024
NEG = -3.4e38


def _rup(a, b):
    return (a + b - 1) // b * b


def _silu(v):
    return v * jax.nn.sigmoid(v)


def _dot(a, b):
    return jnp.dot(a, b, preferred_element_type=jnp.float32)


def _dotx(a, b):
    return jnp.dot(a, b, preferred_element_type=jnp.float32,
                   precision=lax.Precision.HIGHEST)


def _bf(v):
    return v.astype(jnp.bfloat16).astype(jnp.float32)


def _full(shape):
    return pl.BlockSpec(shape, lambda i: (0,) * len(shape))


def _gather(td, di2, si2, e_pad):
    epw = e_pad // NW
    nch = epw // CHUNK
    assert nch % 2 == 0
    mesh = plsc.VectorSubcoreMesh(core_axis_name="c", subcore_axis_name="s",
                                  num_cores=NC, num_subcores=NS)

    def body(td_h, di_h, si_h, gd_h, gs_h, di_v, si_v,
             rd0, rs0, rd1, rs1, g0, g1, w0, w1):
        wid = lax.axis_index("s") * NC + lax.axis_index("c")
        rbase = wid * nch
        ebase = wid * epw
        pltpu.sync_copy(di_h.at[pl.ds(rbase, nch)], di_v)
        pltpu.sync_copy(si_h.at[pl.ds(rbase, nch)], si_v)

        def fire_gather(i, rd, rs, sem):
            pltpu.async_copy(td_h.at[di_v.at[i]], rd, sem)
            pltpu.async_copy(td_h.at[si_v.at[i]], rs, sem)

        def drain2(rd, rs, sem):
            pltpu.make_async_copy(td_h.at[di_v.at[0]], rd, sem).wait()
            pltpu.make_async_copy(td_h.at[si_v.at[0]], rs, sem).wait()

        def fire_write(i, rd, rs, sem):
            off = ebase + i * CHUNK
            pltpu.async_copy(rd, gd_h.at[pl.ds(off, CHUNK)], sem)
            pltpu.async_copy(rs, gs_h.at[pl.ds(off, CHUNK)], sem)

        def drain_w(rd, rs, sem):
            pltpu.make_async_copy(td_h.at[di_v.at[0]], rd, sem).wait()
            pltpu.make_async_copy(td_h.at[di_v.at[0]], rs, sem).wait()

        fire_gather(0, rd0, rs0, g0)

        def step(j, carry):
            c0 = 2 * j
            fire_gather(c0 + 1, rd1, rs1, g1)
            drain2(rd0, rs0, g0)
            fire_write(c0, rd0, rs0, w0)
            drain_w(rd0, rs0, w0)
            fire_gather(c0 + 2, rd0, rs0, g0)
            drain2(rd1, rs1, g1)
            fire_write(c0 + 1, rd1, rs1, w1)
            drain_w(rd1, rs1, w1)
            return carry

        lax.fori_loop(0, nch // 2 - 1, step, 0)
        c0 = nch - 2
        fire_gather(c0 + 1, rd1, rs1, g1)
        drain2(rd0, rs0, g0)
        fire_write(c0, rd0, rs0, w0)
        drain2(rd1, rs1, g1)
        fire_write(c0 + 1, rd1, rs1, w1)
        drain_w(rd0, rs0, w0)
        drain_w(rd1, rs1, w1)

    call = pl.kernel(
        body,
        out_type=(jax.ShapeDtypeStruct((e_pad, TW), jnp.float32),
                  jax.ShapeDtypeStruct((e_pad, TW), jnp.float32)),
        mesh=mesh,
        scratch_types=[
            pltpu.VMEM((nch, CHUNK), jnp.int32),
            pltpu.VMEM((nch, CHUNK), jnp.int32),
            pltpu.VMEM((CHUNK, TW), jnp.float32),
            pltpu.VMEM((CHUNK, TW), jnp.float32),
            pltpu.VMEM((CHUNK, TW), jnp.float32),
            pltpu.VMEM((CHUNK, TW), jnp.float32),
            pltpu.SemaphoreType.DMA,
            pltpu.SemaphoreType.DMA,
            pltpu.SemaphoreType.DMA,
            pltpu.SemaphoreType.DMA,
        ],
        compiler_params=pltpu.CompilerParams(use_tc_tiling_on_sc=False),
    )
    return call(td, di2, si2)


def _scatter(s_arr, dsc2, ztile, e_pad, n_pad):
    epw = e_pad // NW
    nch = epw // CHUNK
    GRP = 2
    SUB = GRP * CHUNK
    ngr = epw // SUB
    per_tile = n_pad // NS
    nzc = per_tile // CHUNK
    mesh = plsc.VectorSubcoreMesh(core_axis_name="c", subcore_axis_name="s",
                                  num_cores=NC, num_subcores=NS)

    def body(s_h, idx_h, z_h, out_h, idx_v, rows_v, acc_sh, lsem):
        c = lax.axis_index("c")
        s = lax.axis_index("s")
        wid = s * NC + c
        rbase = wid * nch
        ebase = wid * epw
        zb = rows_v.at[pl.ds(0, CHUNK)]
        pltpu.sync_copy(z_h, zb)
        for k in range(nzc):
            pltpu.sync_copy(zb,
                            acc_sh.at[pl.ds(s * per_tile + k * CHUNK, CHUNK)])
        plsc.subcore_barrier()

        def grp(g, carry):
            off = ebase + g * SUB
            pltpu.sync_copy(idx_h.at[pl.ds(rbase + g * GRP, GRP)], idx_v)
            pltpu.async_copy(s_h.at[pl.ds(off, SUB)], rows_v, lsem).wait()
            for b in range(GRP):
                pltpu.sync_copy(rows_v.at[pl.ds(b * CHUNK, CHUNK)],
                                acc_sh.at[idx_v.at[b]], add=True)
            return carry

        lax.fori_loop(0, ngr, grp, 0)
        plsc.subcore_barrier()
        wb = rows_v.at[pl.ds(0, CHUNK)]
        for k in range(nzc):
            rof = s * per_tile + k * CHUNK
            pltpu.sync_copy(acc_sh.at[pl.ds(rof, CHUNK)], wb)
            pltpu.sync_copy(wb, out_h.at[c, pl.ds(rof, CHUNK)])

    call = pl.kernel(
        body,
        out_type=jax.ShapeDtypeStruct((NC, n_pad, TW), jnp.float32),
        mesh=mesh,
        scratch_types=[
            pltpu.VMEM((GRP, CHUNK), jnp.int32),
            pltpu.VMEM((SUB, TW), jnp.float32),
            pltpu.VMEM_SHARED((n_pad, TW), jnp.float32),
            pltpu.SemaphoreType.DMA,
        ],
        compiler_params=pltpu.CompilerParams(use_tc_tiling_on_sc=False),
    )
    return call(s_arr, dsc2, ztile)


def _entry(xp, posp, curvp, w_in, b_in, n_pad):
    nin = xp.shape[1]
    cd = curvp.shape[1]

    def body(x_r, pos_r, curv_r, win_r, bin_r, h_r, td_r):
        h = _silu(_dot(x_r[...], win_r[...]) + bin_r[...])
        z = jnp.zeros((BN, TW - H - 3 - cd), jnp.float32)
        h_r[...] = h
        td_r[...] = jnp.concatenate([h, pos_r[...], curv_r[...], z], axis=1)

    return pl.pallas_call(
        body,
        grid=(n_pad // BN,),
        in_specs=[
            pl.BlockSpec((BN, nin), lambda i: (i, 0)),
            pl.BlockSpec((BN, 3), lambda i: (i, 0)),
            pl.BlockSpec((BN, cd), lambda i: (i, 0)),
            _full((nin, H)), _full((1, H)),
        ],
        out_specs=[
            pl.BlockSpec((BN, H), lambda i: (i, 0)),
            pl.BlockSpec((BN, TW), lambda i: (i, 0)),
        ],
        out_shape=[
            jax.ShapeDtypeStruct((n_pad, H), jnp.float32),
            jax.ShapeDtypeStruct((n_pad, TW), jnp.float32),
        ],
    )(xp, posp, curvp, w_in, b_in)


def _edge(gd, gs, we1, wc, we2, wx, be1, be2, bc, bx, cd, e_pad):
    k1 = 2 * H + 2

    def body(gd_r, gs_r, we1_r, wc_r, we2_r, wx_r, be1_r, be2_r, bc_r, bx_r,
             out_r):
        gdv = gd_r[...]
        gsv = gs_r[...]
        hi = gdv[:, :H]
        hj = gsv[:, :H]
        diff = gdv[:, H:H + 3] - gsv[:, H:H + 3]
        d2 = jnp.sum(diff * diff, axis=1, keepdims=True)
        cwcat = jnp.concatenate([gdv[:, H + 3:H + 3 + cd],
                                 gsv[:, H + 3:H + 3 + cd]], axis=1)
        cw = jax.nn.sigmoid(_dot(cwcat, wc_r[...]) + bc_r[0, 0])
        ebig = jnp.concatenate([hi, hj, d2, cw], axis=1)
        m1 = _silu(_dot(ebig, we1_r[...]) + be1_r[...])
        m2 = _silu(_dot(m1, we2_r[...]) + be2_r[...]) * cw
        coef = jnp.tanh(_dot(m2, wx_r[...]) + bx_r[0, 0])
        dx = diff * coef
        ones = jnp.ones((BE, 1), jnp.float32)
        z = jnp.zeros((BE, TW - H - 4), jnp.float32)
        out_r[...] = jnp.concatenate([m2, dx, ones, z], axis=1)

    return pl.pallas_call(
        body,
        grid=(e_pad // BE,),
        in_specs=[
            pl.BlockSpec((BE, TW), lambda i: (i, 0)),
            pl.BlockSpec((BE, TW), lambda i: (i, 0)),
            _full((k1, H)), _full((2 * cd, 1)), _full((H, H)), _full((H, 1)),
            _full((1, H)), _full((1, H)), _full((1, 1)), _full((1, 1)),
        ],
        out_specs=pl.BlockSpec((BE, TW), lambda i: (i, 0)),
        out_shape=jax.ShapeDtypeStruct((e_pad, TW), jnp.float32),
    )(gd, gs, we1, wc, we2, wx, be1, be2, bc, bx)


def _node(l, last, h, acc, xo, deg, curvp, wh1, bh1, wh2, bh2, n_pad):
    cd = curvp.shape[1]

    def body(*refs):
        it = iter(refs)
        h_r = next(it)
        acc_r = next(it)
        xo_r = next(it)
        deg_r = None if l == 0 else next(it)
        curv_r = next(it) if not last else None
        wh1_r = next(it)
        bh1_r = next(it)
        wh2_r = next(it)
        bh2_r = next(it)
        ho_r = next(it)
        xoo_r = next(it)
        dego_r = next(it) if l == 0 else None
        td_r = next(it) if not last else None

        a0 = acc_r[0]
        a1 = acc_r[1]
        agg = a0[:, :H] + a1[:, :H]
        sdx = a0[:, H:H + 3] + a1[:, H:H + 3]
        if l == 0:
            degv = a0[:, H + 3:H + 4] + a1[:, H + 3:H + 4] + 1.0
        else:
            degv = deg_r[...]
        xon = xo_r[...] + sdx / degv
        hv = h_r[...]
        hcat = jnp.concatenate([hv, agg], axis=1)
        upd = _silu(_dot(hcat, wh1_r[...]) + bh1_r[...])
        hn = hv + _dot(upd, wh2_r[...]) + bh2_r[...]
        ho_r[...] = hn
        xoo_r[...] = xon
        if l == 0:
            dego_r[...] = degv
        if not last:
            z = jnp.zeros((BN, TW - H - 3 - cd), jnp.float32)
            td_r[...] = jnp.concatenate([hn, xon, curv_r[...], z], axis=1)

    in_specs = [
        pl.BlockSpec((BN, H), lambda i: (i, 0)),
        pl.BlockSpec((NC, BN, TW), lambda i: (0, i, 0)),
        pl.BlockSpec((BN, 3), lambda i: (i, 0)),
    ]
    args = [h, acc, xo]
    if l != 0:
        in_specs.append(pl.BlockSpec((BN, 1), lambda i: (i, 0)))
        args.append(deg)
    if not last:
        in_specs.append(pl.BlockSpec((BN, cd), lambda i: (i, 0)))
        args.append(curvp)
    in_specs += [_full((2 * H, H)), _full((1, H)), _full((H, H)),
                 _full((1, H))]
    args += [wh1, bh1, wh2, bh2]

    out_specs = [pl.BlockSpec((BN, H), lambda i: (i, 0)),
                 pl.BlockSpec((BN, 3), lambda i: (i, 0))]
    out_shape = [jax.ShapeDtypeStruct((n_pad, H), jnp.float32),
                 jax.ShapeDtypeStruct((n_pad, 3), jnp.float32)]
    if l == 0:
        out_specs.append(pl.BlockSpec((BN, 1), lambda i: (i, 0)))
        out_shape.append(jax.ShapeDtypeStruct((n_pad, 1), jnp.float32))
    if not last:
        out_specs.append(pl.BlockSpec((BN, TW), lambda i: (i, 0)))
        out_shape.append(jax.ShapeDtypeStruct((n_pad, TW), jnp.float32))

    return pl.pallas_call(
        body, grid=(n_pad // BN,), in_specs=in_specs,
        out_specs=out_specs, out_shape=out_shape)(*args)


def _head(h, xo, batch2d, w_out, b_out, w_attn, b_attn, w_mu, b_mu, w_lv, b_lv,
          n_pad):
    aug = H + 1
    nblk = n_pad // BN
    cdims = (((0,), (0,)), ((), ()))

    def _seg(oh, v):
        return lax.dot_general(oh, v, cdims,
                               preferred_element_type=jnp.float32,
                               precision=lax.Precision.HIGHEST)

    def body(h_r, xo_r, b_r, wout_r, bout_r, wattn_r, battn_r, wmu_r, bmu_r,
             wlv_r, blv_r, mu_r, lv_r, cnt_s, xsum_s, gsum_s, gmx_s, den_s,
             ap_s):
        s = pl.program_id(0)
        i = pl.program_id(1)
        bt = b_r[...]
        gi = lax.broadcasted_iota(jnp.int32, (BN, G), 1)
        oh = (bt == gi).astype(jnp.float32)
        valid = bt < G
        xov = xo_r[...]

        @pl.when(jnp.logical_and(s == 0, i == 0))
        def _init():
            cnt_s[...] = jnp.zeros((G, 1), jnp.float32)
            xsum_s[...] = jnp.zeros((G, 3), jnp.float32)
            gsum_s[...] = jnp.zeros((G, aug), jnp.float32)
            gmx_s[...] = jnp.full((G, aug + 1), NEG, jnp.float32)
            den_s[...] = jnp.zeros((G, 1), jnp.float32)
            ap_s[...] = jnp.zeros((G, aug), jnp.float32)

        @pl.when(s == 0)
        def _s0():
            cnt_s[...] += _seg(oh, jnp.ones((BN, 1), jnp.float32))
            xsum_s[...] += _seg(oh, xov)

        def _haug_logits():
            counts = jnp.maximum(cnt_s[...], 1.0)
            cent = xsum_s[...] / counts
            cpn = _dotx(oh, cent)
            dv = xov - cpn
            dist = jnp.sqrt(jnp.sum(dv * dv, axis=1, keepdims=True))
            hout = _dot(h_r[...], wout_r[...]) + bout_r[...]
            haug = jnp.concatenate([hout, dist], axis=1)
            logits = _dot(haug, wattn_r[...]) + battn_r[0, 0]
            return haug, logits

        @pl.when(s == 1)
        def _s1():
            haug, logits = _haug_logits()
            gsum_s[...] += _seg(oh, haug)
            haug2 = jnp.concatenate([haug, logits], axis=1)

            def gmax_step(g, carry):
                mask = bt == g
                mx = jnp.max(jnp.where(mask, haug2, NEG), axis=0,
                             keepdims=True)
                gmx_s[pl.ds(g, 1), :] = jnp.maximum(gmx_s[pl.ds(g, 1), :], mx)
                return carry

            lax.fori_loop(0, G, gmax_step, 0)

        @pl.when(s == 2)
        def _s2():
            _, logits = _haug_logits()
            lpn = _dotx(oh, gmx_s[:, aug:aug + 1])
            ex = jnp.where(valid, jnp.exp(logits - lpn), 0.0)
            den_s[...] += _seg(oh, ex)

        @pl.when(s == 3)
        def _s3():
            haug, logits = _haug_logits()
            lpn = _dotx(oh, gmx_s[:, aug:aug + 1])
            ex = jnp.where(valid, jnp.exp(logits - lpn), 0.0)
            dpn = _dotx(oh, den_s[...])
            attn = jnp.where(valid, ex / dpn, 0.0)
            ap_s[...] += _seg(oh, attn * haug)

        @pl.when(jnp.logical_and(s == 3, i == nblk - 1))
        def _fin():
            counts = jnp.maximum(cnt_s[...], 1.0)
            gmean = gsum_s[...] / counts
            pooled = jnp.concatenate([gmean, gmx_s[:, :aug], ap_s[...]],
                                     axis=1)
            mu_r[...] = _dot(pooled, wmu_r[...]) + bmu_r[...]
            lv_r[...] = jnp.clip(_dot(pooled, wlv_r[...]) + blv_r[...],
                                 -10.0, 10.0)

    lat = w_mu.shape[1]
    blk = lambda w: pl.BlockSpec((BN, w), lambda s, i: (i, 0))
    fullg = lambda shp: pl.BlockSpec(shp, lambda s, i: (0,) * len(shp))
    return pl.pallas_call(
        body,
        grid=(4, nblk),
        in_specs=[
            blk(H), blk(3), blk(1),
            fullg((H, H)), fullg((1, H)), fullg((aug, 1)), fullg((1, 1)),
            fullg((3 * aug, lat)), fullg((1, lat)), fullg((3 * aug, lat)),
            fullg((1, lat)),
        ],
        out_specs=[pl.BlockSpec((G, lat), lambda s, i: (0, 0)),
                   pl.BlockSpec((G, lat), lambda s, i: (0, 0))],
        scratch_shapes=[
            pltpu.VMEM((G, 1), jnp.float32),
            pltpu.VMEM((G, 3), jnp.float32),
            pltpu.VMEM((G, aug), jnp.float32),
            pltpu.VMEM((G, aug + 1), jnp.float32),
            pltpu.VMEM((G, 1), jnp.float32),
            pltpu.VMEM((G, aug), jnp.float32),
        ],
        out_shape=[jax.ShapeDtypeStruct((G, lat), jnp.float32),
                   jax.ShapeDtypeStruct((G, lat), jnp.float32)],
    )(h, xo, batch2d, w_out, b_out, w_attn, b_attn, w_mu, b_mu, w_lv, b_lv)


def kernel(x, pos, curv, edge_index, batch, W_in, b_in, We1, be1, We2, be2,
           Wc, bc, Wx, bx, Wh1, bh1, Wh2, bh2, W_out, b_out, W_attn, b_attn,
           W_mu, b_mu, W_lv, b_lv):
    n = x.shape[0]
    e = edge_index.shape[1]
    nl = We1.shape[0]
    cd = curv.shape[1]
    n_pad = _rup(n, NS * CHUNK)
    e_pad = _rup(e, NW * CHUNK)

    xp = jnp.pad(x, ((0, n_pad - n), (0, 0)))
    posp = jnp.pad(pos, ((0, n_pad - n), (0, 0)))
    curvp = jnp.pad(curv, ((0, n_pad - n), (0, 0)))
    batch2d = jnp.pad(batch, (0, n_pad - n),
                      constant_values=G).reshape(n_pad, 1)
    src = edge_index[0]
    dst = edge_index[1]
    dig = jnp.pad(dst, (0, e_pad - e)).reshape(e_pad // CHUNK, CHUNK)
    sig = jnp.pad(src, (0, e_pad - e)).reshape(e_pad // CHUNK, CHUNK)
    dsc = jnp.pad(dst, (0, e_pad - e),
                  constant_values=n_pad - 1).reshape(e_pad // CHUNK, CHUNK)
    ztile = jnp.zeros((CHUNK, TW), jnp.float32)

    h, td = _entry(xp, posp, curvp, W_in, b_in.reshape(1, H), n_pad)
    xo = posp
    deg = None
    for l in range(nl):
        last = l == nl - 1
        gd, gs = _gather(td, dig, sig, e_pad)
        s_arr = _edge(gd, gs, We1[l], Wc[l], We2[l], Wx[l],
                      be1[l].reshape(1, H), be2[l].reshape(1, H),
                      bc[l].reshape(1, 1), bx[l].reshape(1, 1), cd, e_pad)
        acc = _scatter(s_arr, dsc, ztile, e_pad, n_pad)
        outs = _node(l, last, h, acc, xo, deg, curvp,
                     Wh1[l], bh1[l].reshape(1, H),
                     Wh2[l], bh2[l].reshape(1, H), n_pad)
        it = iter(outs)
        h = next(it)
        xo = next(it)
        if l == 0:
            deg = next(it)
        if not last:
            td = next(it)

    mu, lv = _head(h, xo, batch2d, W_out, b_out.reshape(1, H), W_attn,
                   b_attn.reshape(1, 1), W_mu, b_mu.reshape(1, -1), W_lv,
                   b_lv.reshape(1, -1), n_pad)
    return (mu, lv)
